# Initial kernel scaffold; baseline (speedup 1.0000x reference)
import jax
import jax.numpy as jnp
from jax.experimental import pallas as pl


def kernel(context, target, neg_targets, W_in, W_out):
    raise NotImplementedError("write your pallas kernel here")


# speedup vs baseline: 4.7898x; 4.7898x over previous
import functools

import jax
import jax.numpy as jnp
from jax import lax
from jax.experimental import pallas as pl
from jax.experimental.pallas import tpu as pltpu
from jax.experimental.pallas import tpu_sc as plsc


@functools.lru_cache(maxsize=None)
def _make_scores_kernel(V, D, B, CTX, K):
    info = plsc.get_sparse_core_info()
    NC, NS, L = info.num_cores, info.num_subcores, info.num_lanes
    NW = NC * NS
    E = B // NW
    CB = 32
    NCH = E // CB
    DB = D // L
    G = CB // L
    NCTX = CB * CTX
    NNEG = CB * K
    GLEN = 128
    mesh = plsc.VectorSubcoreMesh(core_axis_name="c", subcore_axis_name="s")

    @functools.partial(
        pl.kernel,
        out_type=[
            jax.ShapeDtypeStruct((B,), jnp.float32),
            jax.ShapeDtypeStruct((B * K,), jnp.float32),
        ],
        mesh=mesh,
        compiler_params=pltpu.CompilerParams(needs_layout_passes=False, use_tc_tiling_on_sc=False),
        scratch_types=[
            pltpu.VMEM((NCTX,), jnp.int32),
            pltpu.VMEM((NNEG,), jnp.int32),
            pltpu.VMEM((CB,), jnp.int32),
            pltpu.VMEM((NCTX, D), jnp.float32),
            pltpu.VMEM((NNEG, D), jnp.float32),
            pltpu.VMEM((CB, D), jnp.float32),
            pltpu.VMEM((CB, D), jnp.float32),
            pltpu.VMEM((CB,), jnp.float32),
            pltpu.VMEM((NNEG,), jnp.float32),
            pltpu.VMEM((L * L,), jnp.float32),
            pltpu.SemaphoreType.DMA,
            pltpu.SemaphoreType.DMA,
            pltpu.SemaphoreType.DMA,
        ],
    )
    def scores_kernel(ctx_hbm, tgt_hbm, neg_hbm, win_hbm, wout_hbm,
                      pos_out, negs_out,
                      ctx_idx, neg_idx, tgt_idx, ctx_rows, neg_rows,
                      tgt_rows, h_v, pos_v, negs_v, tsc,
                      sem_c, sem_t, sem_n):
        wid = lax.axis_index("s") * NC + lax.axis_index("c")
        iota = lax.iota(jnp.int32, L)

        def chunk_body(c, carry):
            base = wid * E + c * CB
            pltpu.sync_copy(ctx_hbm.at[pl.ds(base * CTX, NCTX)], ctx_idx)
            pltpu.sync_copy(neg_hbm.at[pl.ds(base * K, NNEG)], neg_idx)
            pltpu.sync_copy(tgt_hbm.at[pl.ds(base, CB)], tgt_idx)
            ctx_cps = [
                pltpu.async_copy(
                    win_hbm.at[ctx_idx.at[pl.ds(q * GLEN, GLEN)]],
                    ctx_rows.at[pl.ds(q * GLEN, GLEN)], sem_c)
                for q in range(NCTX // GLEN)
            ]
            tgt_cp = pltpu.async_copy(wout_hbm.at[tgt_idx], tgt_rows, sem_t)
            neg_cps = [
                pltpu.async_copy(
                    wout_hbm.at[neg_idx.at[pl.ds(q * GLEN, GLEN)]],
                    neg_rows.at[pl.ds(q * GLEN, GLEN)], sem_n)
                for q in range(NNEG // GLEN)
            ]
            for cp in ctx_cps:
                cp.wait()

            def h_body(b, carry2):
                for kb in range(DB):
                    acc = ctx_rows[b * CTX, pl.ds(kb * L, L)]
                    for j in range(1, CTX):
                        acc = acc + ctx_rows[b * CTX + j, pl.ds(kb * L, L)]
                    h_v[b, pl.ds(kb * L, L)] = acc * (1.0 / CTX)
                return carry2
            lax.fori_loop(0, CB, h_body, 0)

            tgt_cp.wait()
            for g in range(G):
                for b16 in range(L):
                    b = g * L + b16
                    p = tgt_rows[b, pl.ds(0, L)] * h_v[b, pl.ds(0, L)]
                    for kb in range(1, DB):
                        p = p + (tgt_rows[b, pl.ds(kb * L, L)]
                                 * h_v[b, pl.ds(kb * L, L)])
                    plsc.store_scatter(tsc, [iota * L + b16], p)
                s = tsc[pl.ds(0, L)]
                for i in range(1, L):
                    s = s + tsc[pl.ds(i * L, L)]
                pos_v[pl.ds(g * L, L)] = s

            for cp in neg_cps:
                cp.wait()

            def neg_body(j, carry2):
                for g in range(G):
                    for b16 in range(L):
                        b = g * L + b16
                        r = b * K + j
                        p = neg_rows[r, pl.ds(0, L)] * h_v[b, pl.ds(0, L)]
                        for kb in range(1, DB):
                            p = p + (neg_rows[r, pl.ds(kb * L, L)]
                                     * h_v[b, pl.ds(kb * L, L)])
                        plsc.store_scatter(tsc, [iota * L + b16], p)
                    s = tsc[pl.ds(0, L)]
                    for i in range(1, L):
                        s = s + tsc[pl.ds(i * L, L)]
                    plsc.store_scatter(
                        negs_v, [(iota + g * L) * K + j], s)
                return carry2
            lax.fori_loop(0, K, neg_body, 0)

            pltpu.sync_copy(pos_v, pos_out.at[pl.ds(base, CB)])
            pltpu.sync_copy(negs_v, negs_out.at[pl.ds(base * K, NNEG)])
            return carry

        lax.fori_loop(0, NCH, chunk_body, 0)

    return scores_kernel


@functools.lru_cache(maxsize=None)
def _make_loss_kernel(B):
    def loss_body(pos_ref, neg_ref, out_ref):
        p = pos_ref[...]
        n = neg_ref[...]
        ls_p = jnp.minimum(p, 0.0) - jnp.log1p(jnp.exp(-jnp.abs(p)))
        ls_n = jnp.minimum(-n, 0.0) - jnp.log1p(jnp.exp(-jnp.abs(n)))
        out_ref[0] = -(jnp.sum(ls_p) + jnp.sum(ls_n)) / B

    return pl.pallas_call(
        loss_body,
        out_shape=jax.ShapeDtypeStruct((1,), jnp.float32),
        out_specs=pl.BlockSpec(memory_space=pltpu.SMEM),
    )


def kernel(context, target, neg_targets, W_in, W_out):
    B, CTX = context.shape
    K = neg_targets.shape[1]
    V, D = W_in.shape
    ctx_flat = context.astype(jnp.int32).reshape(-1)
    neg_flat = neg_targets.astype(jnp.int32).reshape(-1)
    tgt = target.astype(jnp.int32)
    pos_s, neg_s = _make_scores_kernel(V, D, B, CTX, K)(
        ctx_flat, tgt, neg_flat, W_in, W_out)
    loss = _make_loss_kernel(B)(
        pos_s.reshape(B // 128, 128), neg_s.reshape(K * B // 128, 128))
    return loss[0]
